# baseline (speedup 1.0000x reference)
import functools
import math

import jax
import jax.numpy as jnp
from jax.experimental import pallas as pl
from jax.experimental.pallas import tpu as pltpu

B, N = 1, 2048
D, H = 768, 12
HD = D // H
E, MULT = 8, 4
F = MULT * D
VOCAB, NPERT, LORA = 65, 10000, 32
MAXFREQ = 64
NDENSE, NMOE = 2, 2

T_MOE = 256
NT = (2 * N) // T_MOE + E

_INTERPRET = False


def _gelu(x):
    return jax.nn.gelu(x, approximate=True)


def _ln_in(x, g, b):
    m = jnp.mean(x, axis=-1, keepdims=True)
    xm = x - m
    v = jnp.mean(xm * xm, axis=-1, keepdims=True)
    return xm * jax.lax.rsqrt(v + 1e-5) * g + b


def _embed_body(tok_ref, tab_ref, fourier_ref, tw_ref, tb_ref, pert_ref,
                la_ref, lb_ref, out_ref):
    bias = (jnp.dot(fourier_ref[...], tw_ref[...],
                    preferred_element_type=jnp.float32)
            + tb_ref[...] + pert_ref[...]
            + jnp.dot(la_ref[...], lb_ref[...],
                      preferred_element_type=jnp.float32))
    lanes = jax.lax.broadcasted_iota(jnp.int32, (N, 128), 1)
    onehot = (tok_ref[...] == lanes).astype(jnp.float32)
    out_ref[...] = jnp.dot(onehot, tab_ref[...],
                           preferred_element_type=jnp.float32) + bias


def _embed(tokens, table_p, fourier, time_W, time_b, pert_row, la_row, loraB):
    return pl.pallas_call(
        _embed_body,
        out_shape=jax.ShapeDtypeStruct((N, D), jnp.float32),
        interpret=_INTERPRET,
    )(tokens, table_p, fourier, time_W, time_b, pert_row, la_row, loraB)


def _ln_qkv_body(x_ref, g_ref, b_ref, w_ref, out_ref):
    h = _ln_in(x_ref[...], g_ref[...], b_ref[...]).astype(jnp.bfloat16)
    out_ref[...] = jnp.dot(h, w_ref[...],
                           preferred_element_type=jnp.float32
                           ).astype(jnp.bfloat16)


def _ln_qkv(x, g, b, w):
    TM = 256
    return pl.pallas_call(
        _ln_qkv_body,
        grid=(N // TM,),
        in_specs=[
            pl.BlockSpec((TM, D), lambda i: (i, 0)),
            pl.BlockSpec((1, D), lambda i: (0, 0)),
            pl.BlockSpec((1, D), lambda i: (0, 0)),
            pl.BlockSpec((D, 3 * D), lambda i: (0, 0)),
        ],
        out_specs=pl.BlockSpec((TM, 3 * D), lambda i: (i, 0)),
        out_shape=jax.ShapeDtypeStruct((N, 3 * D), jnp.bfloat16),
        interpret=_INTERPRET,
    )(x, g, b, w)


def _attn_body(q_ref, k_ref, v_ref, o_ref):
    q = q_ref[0]
    s = jax.lax.dot_general(q, k_ref[0], (((1,), (1,)), ((), ())),
                            preferred_element_type=jnp.float32)
    s = s * (HD ** -0.5)
    m = jnp.max(s, axis=1, keepdims=True)
    p = jnp.exp(s - m)
    l = jnp.sum(p, axis=1, keepdims=True)
    o = jnp.dot(p.astype(jnp.bfloat16), v_ref[0],
                preferred_element_type=jnp.float32)
    o_ref[0] = (o / l).astype(jnp.bfloat16)


def _attention(qkv):
    qkv_t = qkv.reshape(N, 3 * H, HD).transpose(1, 0, 2)
    TQ = 1024
    o = pl.pallas_call(
        _attn_body,
        grid=(H, N // TQ),
        in_specs=[
            pl.BlockSpec((1, TQ, HD), lambda h, i: (h, i, 0)),
            pl.BlockSpec((1, N, HD), lambda h, i: (H + h, 0, 0)),
            pl.BlockSpec((1, N, HD), lambda h, i: (2 * H + h, 0, 0)),
        ],
        out_specs=pl.BlockSpec((1, TQ, HD), lambda h, i: (h, i, 0)),
        out_shape=jax.ShapeDtypeStruct((H, N, HD), jnp.bfloat16),
        interpret=_INTERPRET,
    )(qkv_t, qkv_t, qkv_t)
    return o.transpose(1, 0, 2).reshape(N, D)


def _proj_mlp_body(x_ref, o_ref, pw_ref, g_ref, b_ref, fc1_ref, b1_ref,
                   fc2_ref, b2_ref, out_ref):
    x2 = x_ref[...] + jnp.dot(o_ref[...], pw_ref[...],
                              preferred_element_type=jnp.float32)
    h2 = _ln_in(x2, g_ref[...], b_ref[...]).astype(jnp.bfloat16)
    a = _gelu(jnp.dot(h2, fc1_ref[...], preferred_element_type=jnp.float32)
              + b1_ref[...]).astype(jnp.bfloat16)
    y = jnp.dot(a, fc2_ref[...], preferred_element_type=jnp.float32) + b2_ref[...]
    out_ref[...] = x2 + y


def _proj_mlp(x, o, proj, g, b, fc1, b1, fc2, b2):
    TM = 256
    return pl.pallas_call(
        _proj_mlp_body,
        grid=(N // TM,),
        in_specs=[
            pl.BlockSpec((TM, D), lambda i: (i, 0)),
            pl.BlockSpec((TM, D), lambda i: (i, 0)),
            pl.BlockSpec((D, D), lambda i: (0, 0)),
            pl.BlockSpec((1, D), lambda i: (0, 0)),
            pl.BlockSpec((1, D), lambda i: (0, 0)),
            pl.BlockSpec((D, F), lambda i: (0, 0)),
            pl.BlockSpec((1, F), lambda i: (0, 0)),
            pl.BlockSpec((F, D), lambda i: (0, 0)),
            pl.BlockSpec((1, D), lambda i: (0, 0)),
        ],
        out_specs=pl.BlockSpec((TM, D), lambda i: (i, 0)),
        out_shape=jax.ShapeDtypeStruct((N, D), jnp.float32),
        interpret=_INTERPRET,
    )(x, o, proj, g, b, fc1, b1, fc2, b2)


def _router_body(x_ref, g_ref, b_ref, gate_ref, h_ref, cw_ref, rk_ref,
                 cnt_ref):
    h = _ln_in(x_ref[...], g_ref[...], b_ref[...])
    h_ref[...] = h.astype(jnp.bfloat16)
    lt = jax.lax.dot_general(gate_ref[...], h, (((0,), (1,)), ((), ())),
                             preferred_element_type=jnp.float32)
    m = jnp.max(lt, axis=0, keepdims=True)
    p = jnp.exp(lt - m)
    p = p / jnp.sum(p, axis=0, keepdims=True)
    eio = jax.lax.broadcasted_iota(jnp.int32, (E, N), 0)
    v1 = jnp.max(p, axis=0, keepdims=True)
    i1 = jnp.min(jnp.where(p == v1, eio, E + 1), axis=0, keepdims=True)
    pm = jnp.where(eio == i1, -jnp.inf, p)
    v2 = jnp.max(pm, axis=0, keepdims=True)
    i2 = jnp.min(jnp.where(pm == v2, eio, E + 1), axis=0, keepdims=True)
    denom = v1 + v2 + 1e-9
    cw = (jnp.where(eio == i1, v1 / denom, 0.0)
          + jnp.where(eio == i2, v2 / denom, 0.0))
    cw_ref[...] = cw
    ind = ((eio == i1) | (eio == i2)).astype(jnp.float32)
    acc = ind
    s = 1
    while s < N:
        shifted = jnp.pad(acc, ((0, 0), (s, 0)))[:, :N]
        acc = acc + shifted
        s *= 2
    excl = acc - ind
    rk_ref[...] = jnp.where(ind > 0.5, excl, -1.0)
    cnt_ref[...] = jnp.sum(ind, axis=1, keepdims=True)


def _router(x, g, b, gate):
    return pl.pallas_call(
        _router_body,
        out_shape=(
            jax.ShapeDtypeStruct((N, D), jnp.bfloat16),
            jax.ShapeDtypeStruct((E, N), jnp.float32),
            jax.ShapeDtypeStruct((E, N), jnp.float32),
            jax.ShapeDtypeStruct((E, 1), jnp.float32),
        ),
        interpret=_INTERPRET,
    )(x, g, b, gate)


def _moe_body(map_ref, base_ref, h_ref, cw_ref, rk_ref, w1_ref,
              b1_ref, w2_ref, b2_ref, out_ref):
    j = pl.program_id(0)
    e = map_ref[j]
    base = base_ref[j].astype(jnp.float32)
    rk = rk_ref[pl.ds(e, 1), :]
    cwr = cw_ref[pl.ds(e, 1), :]
    rows = jax.lax.broadcasted_iota(
        jnp.int32, (T_MOE, 1), 0).astype(jnp.float32) + base
    g01 = (rk == rows).astype(jnp.bfloat16)
    ht = jnp.dot(g01, h_ref[...],
                 preferred_element_type=jnp.float32).astype(jnp.bfloat16)
    a = _gelu(jnp.dot(ht, w1_ref[0], preferred_element_type=jnp.float32)
              + b1_ref[0]).astype(jnp.bfloat16)
    y = (jnp.dot(a, w2_ref[0], preferred_element_type=jnp.float32)
         + b2_ref[0]).astype(jnp.bfloat16)
    gw = g01 * cwr.astype(jnp.bfloat16)
    contrib = jax.lax.dot_general(gw, y, (((0,), (0,)), ((), ())),
                                  preferred_element_type=jnp.float32)

    @pl.when(j == 0)
    def _():
        out_ref[...] = contrib

    @pl.when(j != 0)
    def _():
        out_ref[...] = out_ref[...] + contrib


def _moe_dispatch(tile_map, tile_base, h, cw, rk, w1, b1, w2, b2):
    grid_spec = pltpu.PrefetchScalarGridSpec(
        num_scalar_prefetch=2,
        grid=(NT,),
        in_specs=[
            pl.BlockSpec((N, D), lambda j, m, bs: (0, 0)),
            pl.BlockSpec((E, N), lambda j, m, bs: (0, 0)),
            pl.BlockSpec((E, N), lambda j, m, bs: (0, 0)),
            pl.BlockSpec((1, D, F), lambda j, m, bs: (m[j], 0, 0)),
            pl.BlockSpec((1, 1, F), lambda j, m, bs: (m[j], 0, 0)),
            pl.BlockSpec((1, F, D), lambda j, m, bs: (m[j], 0, 0)),
            pl.BlockSpec((1, 1, D), lambda j, m, bs: (m[j], 0, 0)),
        ],
        out_specs=pl.BlockSpec((N, D), lambda j, m, bs: (0, 0)),
    )
    return pl.pallas_call(
        _moe_body,
        grid_spec=grid_spec,
        out_shape=jax.ShapeDtypeStruct((N, D), jnp.float32),
        interpret=_INTERPRET,
    )(tile_map, tile_base, h, cw, rk, w1, b1, w2, b2)


def _moe_block(x, p):
    h, cw, rk, cnt = _router(x, p['ln_g'].reshape(1, D),
                             p['ln_b'].reshape(1, D), p['gate'])
    counts = cnt.reshape(E).astype(jnp.int32)
    tiles = (counts + (T_MOE - 1)) // T_MOE
    tend = jnp.cumsum(tiles)
    tstart = tend - tiles
    j = jnp.arange(NT, dtype=jnp.int32)
    e_of = jnp.sum((j[:, None] >= tend[None, :]).astype(jnp.int32), axis=1)
    valid = e_of < E
    e_clamped = jnp.where(valid, e_of, 0)
    tile_map = e_clamped.astype(jnp.int32)
    tile_base = jnp.where(
        valid, (j - tstart[e_clamped]) * T_MOE, jnp.int32(1 << 20)
    ).astype(jnp.int32)
    return x + _moe_dispatch(tile_map, tile_base, h, cw, rk,
                             p['w1'].astype(jnp.bfloat16),
                             p['b1'].reshape(E, 1, F),
                             p['w2'].astype(jnp.bfloat16),
                             p['b2'].reshape(E, 1, D))


def _head_body(x_ref, g_ref, b_ref, w_ref, hb_ref, out_ref):
    h = _ln_in(x_ref[...], g_ref[...], b_ref[...])
    out_ref[...] = jnp.dot(h, w_ref[...],
                           preferred_element_type=jnp.float32) + hb_ref[...]


def _head(x, g, b, w, hb):
    TM = 256
    VO = VOCAB - 1
    return pl.pallas_call(
        _head_body,
        grid=(N // TM,),
        in_specs=[
            pl.BlockSpec((TM, D), lambda i: (i, 0)),
            pl.BlockSpec((1, D), lambda i: (0, 0)),
            pl.BlockSpec((1, D), lambda i: (0, 0)),
            pl.BlockSpec((D, VO), lambda i: (0, 0)),
            pl.BlockSpec((1, VO), lambda i: (0, 0)),
        ],
        out_specs=pl.BlockSpec((TM, VO), lambda i: (i, 0)),
        out_shape=jax.ShapeDtypeStruct((N, VO), jnp.float32),
        interpret=_INTERPRET,
    )(x, g, b, w, hb)


def _dense_block(x, p):
    qkv = _ln_qkv(x, p['ln1_g'].reshape(1, D), p['ln1_b'].reshape(1, D),
                  p['qkv'].astype(jnp.bfloat16))
    o = _attention(qkv)
    return _proj_mlp(x, o, p['proj'].astype(jnp.bfloat16),
                     p['ln2_g'].reshape(1, D),
                     p['ln2_b'].reshape(1, D), p['fc1'].astype(jnp.bfloat16),
                     p['b1'].reshape(1, F), p['fc2'].astype(jnp.bfloat16),
                     p['b2'].reshape(1, D))


def kernel(tokens, time_t, perturb_id, params):
    freqs = jnp.arange(MAXFREQ, dtype=jnp.float32)
    angles = time_t[:, None] * freqs[None, :] * math.pi
    fourier = jnp.concatenate([jnp.sin(angles), jnp.cos(angles)], axis=-1)
    table_p = jnp.pad(params['token_emb'], ((0, 128 - VOCAB), (0, 0)))
    pert_row = params['pert_base'][perturb_id]
    la_row = params['loraA'][perturb_id]
    tok2d = tokens.reshape(N, 1).astype(jnp.int32)
    x = _embed(tok2d, table_p, fourier, params['time_W'],
               params['time_b'].reshape(1, D), pert_row, la_row,
               params['loraB'])
    for i in range(max(NDENSE, NMOE)):
        if i < NDENSE:
            x = _dense_block(x, params['dense'][i])
        if i < NMOE:
            x = _moe_block(x, params['moe'][i])
    out = _head(x, params['lnf_g'].reshape(1, D), params['lnf_b'].reshape(1, D),
                params['head_W'], params['head_b'].reshape(1, VOCAB - 1))
    return out.reshape(B, N, VOCAB - 1)


# speedup vs baseline: 1.0068x; 1.0068x over previous
import functools
import math

import jax
import jax.numpy as jnp
from jax.experimental import pallas as pl
from jax.experimental.pallas import tpu as pltpu

B, N = 1, 2048
D, H = 768, 12
HD = D // H
E, MULT = 8, 4
F = MULT * D
VOCAB, NPERT, LORA = 65, 10000, 32
MAXFREQ = 64
NDENSE, NMOE = 2, 2

T_MOE = 128
NT = (2 * N) // T_MOE + E

_INTERPRET = False


def _gelu(x):
    return jax.nn.gelu(x, approximate=True)


def _ln_in(x, g, b):
    m = jnp.mean(x, axis=-1, keepdims=True)
    xm = x - m
    v = jnp.mean(xm * xm, axis=-1, keepdims=True)
    return xm * jax.lax.rsqrt(v + 1e-5) * g + b


def _embed_body(tok_ref, tab_ref, fourier_ref, tw_ref, tb_ref, pert_ref,
                la_ref, lb_ref, out_ref):
    bias = (jnp.dot(fourier_ref[...], tw_ref[...],
                    preferred_element_type=jnp.float32)
            + tb_ref[...] + pert_ref[...]
            + jnp.dot(la_ref[...], lb_ref[...],
                      preferred_element_type=jnp.float32))
    lanes = jax.lax.broadcasted_iota(jnp.int32, (N, 128), 1)
    onehot = (tok_ref[...] == lanes).astype(jnp.float32)
    out_ref[...] = jnp.dot(onehot, tab_ref[...],
                           preferred_element_type=jnp.float32) + bias


def _embed(tokens, table_p, fourier, time_W, time_b, pert_row, la_row, loraB):
    return pl.pallas_call(
        _embed_body,
        out_shape=jax.ShapeDtypeStruct((N, D), jnp.float32),
        interpret=_INTERPRET,
    )(tokens, table_p, fourier, time_W, time_b, pert_row, la_row, loraB)


def _ln_qkv_body(x_ref, g_ref, b_ref, w_ref, out_ref):
    h = _ln_in(x_ref[...], g_ref[...], b_ref[...]).astype(jnp.bfloat16)
    out_ref[...] = jnp.dot(h, w_ref[...],
                           preferred_element_type=jnp.float32
                           ).astype(jnp.bfloat16)


def _ln_qkv(x, g, b, w):
    TM = 256
    return pl.pallas_call(
        _ln_qkv_body,
        grid=(N // TM,),
        in_specs=[
            pl.BlockSpec((TM, D), lambda i: (i, 0)),
            pl.BlockSpec((1, D), lambda i: (0, 0)),
            pl.BlockSpec((1, D), lambda i: (0, 0)),
            pl.BlockSpec((D, 3 * D), lambda i: (0, 0)),
        ],
        out_specs=pl.BlockSpec((TM, 3 * D), lambda i: (i, 0)),
        out_shape=jax.ShapeDtypeStruct((N, 3 * D), jnp.bfloat16),
        interpret=_INTERPRET,
    )(x, g, b, w)


def _attn_body(q_ref, k_ref, v_ref, o_ref):
    q = (q_ref[0].astype(jnp.float32) * (HD ** -0.5)).astype(jnp.bfloat16)
    s = jax.lax.dot_general(q, k_ref[0], (((1,), (1,)), ((), ())),
                            preferred_element_type=jnp.float32)
    m = jnp.max(s, axis=1, keepdims=True)
    p = jnp.exp((s - m).astype(jnp.bfloat16))
    vp = jnp.concatenate(
        [v_ref[0], jnp.ones((N, HD), jnp.bfloat16)], axis=1)
    ol = jnp.dot(p, vp, preferred_element_type=jnp.float32)
    o_ref[0] = (ol[:, :HD] / ol[:, HD:HD + 1]).astype(jnp.bfloat16)


def _attention(qkv):
    qkv_t = qkv.reshape(N, 3 * H, HD).transpose(1, 0, 2)
    o = pl.pallas_call(
        _attn_body,
        grid=(H,),
        in_specs=[
            pl.BlockSpec((1, N, HD), lambda h: (h, 0, 0)),
            pl.BlockSpec((1, N, HD), lambda h: (H + h, 0, 0)),
            pl.BlockSpec((1, N, HD), lambda h: (2 * H + h, 0, 0)),
        ],
        out_specs=pl.BlockSpec((1, N, HD), lambda h: (h, 0, 0)),
        out_shape=jax.ShapeDtypeStruct((H, N, HD), jnp.bfloat16),
        interpret=_INTERPRET,
    )(qkv_t, qkv_t, qkv_t)
    return o.transpose(1, 0, 2).reshape(N, D)


def _proj_mlp_body(x_ref, o_ref, pw_ref, g_ref, b_ref, fc1_ref, b1_ref,
                   fc2_ref, b2_ref, out_ref):
    x2 = x_ref[...] + jnp.dot(o_ref[...], pw_ref[...],
                              preferred_element_type=jnp.float32)
    h2 = _ln_in(x2, g_ref[...], b_ref[...]).astype(jnp.bfloat16)
    a = _gelu(jnp.dot(h2, fc1_ref[...], preferred_element_type=jnp.float32)
              + b1_ref[...]).astype(jnp.bfloat16)
    y = jnp.dot(a, fc2_ref[...], preferred_element_type=jnp.float32) + b2_ref[...]
    out_ref[...] = x2 + y


def _proj_mlp(x, o, proj, g, b, fc1, b1, fc2, b2):
    TM = 256
    return pl.pallas_call(
        _proj_mlp_body,
        grid=(N // TM,),
        in_specs=[
            pl.BlockSpec((TM, D), lambda i: (i, 0)),
            pl.BlockSpec((TM, D), lambda i: (i, 0)),
            pl.BlockSpec((D, D), lambda i: (0, 0)),
            pl.BlockSpec((1, D), lambda i: (0, 0)),
            pl.BlockSpec((1, D), lambda i: (0, 0)),
            pl.BlockSpec((D, F), lambda i: (0, 0)),
            pl.BlockSpec((1, F), lambda i: (0, 0)),
            pl.BlockSpec((F, D), lambda i: (0, 0)),
            pl.BlockSpec((1, D), lambda i: (0, 0)),
        ],
        out_specs=pl.BlockSpec((TM, D), lambda i: (i, 0)),
        out_shape=jax.ShapeDtypeStruct((N, D), jnp.float32),
        interpret=_INTERPRET,
    )(x, o, proj, g, b, fc1, b1, fc2, b2)


def _router_body(x_ref, g_ref, b_ref, gate_ref, h_ref, cw_ref, rk_ref,
                 cnt_ref):
    h = _ln_in(x_ref[...], g_ref[...], b_ref[...])
    h_ref[...] = h.astype(jnp.bfloat16)
    lt = jax.lax.dot_general(gate_ref[...], h, (((0,), (1,)), ((), ())),
                             preferred_element_type=jnp.float32)
    m = jnp.max(lt, axis=0, keepdims=True)
    p = jnp.exp(lt - m)
    p = p / jnp.sum(p, axis=0, keepdims=True)
    eio = jax.lax.broadcasted_iota(jnp.int32, (E, N), 0)
    v1 = jnp.max(p, axis=0, keepdims=True)
    i1 = jnp.min(jnp.where(p == v1, eio, E + 1), axis=0, keepdims=True)
    pm = jnp.where(eio == i1, -jnp.inf, p)
    v2 = jnp.max(pm, axis=0, keepdims=True)
    i2 = jnp.min(jnp.where(pm == v2, eio, E + 1), axis=0, keepdims=True)
    denom = v1 + v2 + 1e-9
    cw = (jnp.where(eio == i1, v1 / denom, 0.0)
          + jnp.where(eio == i2, v2 / denom, 0.0))
    cw_ref[...] = cw
    ind = ((eio == i1) | (eio == i2)).astype(jnp.float32)
    acc = ind
    s = 1
    while s < N:
        shifted = jnp.pad(acc, ((0, 0), (s, 0)))[:, :N]
        acc = acc + shifted
        s *= 2
    excl = acc - ind
    rk_ref[...] = jnp.where(ind > 0.5, excl, -1.0)
    cnt_ref[...] = jnp.sum(ind, axis=1, keepdims=True)


def _router(x, g, b, gate):
    return pl.pallas_call(
        _router_body,
        out_shape=(
            jax.ShapeDtypeStruct((N, D), jnp.bfloat16),
            jax.ShapeDtypeStruct((E, N), jnp.float32),
            jax.ShapeDtypeStruct((E, N), jnp.float32),
            jax.ShapeDtypeStruct((E, 1), jnp.float32),
        ),
        interpret=_INTERPRET,
    )(x, g, b, gate)


def _moe_body(map_ref, base_ref, h_ref, cw_ref, rk_ref, w1_ref,
              b1_ref, w2_ref, b2_ref, out_ref):
    j = pl.program_id(0)
    e = map_ref[j]
    base = base_ref[j].astype(jnp.float32)
    rk = rk_ref[pl.ds(e, 1), :]
    cwr = cw_ref[pl.ds(e, 1), :]
    rows = jax.lax.broadcasted_iota(
        jnp.int32, (T_MOE, 1), 0).astype(jnp.float32) + base
    g01 = (rk == rows).astype(jnp.bfloat16)
    ht = jnp.dot(g01, h_ref[...],
                 preferred_element_type=jnp.float32).astype(jnp.bfloat16)
    a = _gelu(jnp.dot(ht, w1_ref[0], preferred_element_type=jnp.float32)
              + b1_ref[0]).astype(jnp.bfloat16)
    y = (jnp.dot(a, w2_ref[0], preferred_element_type=jnp.float32)
         + b2_ref[0]).astype(jnp.bfloat16)
    gw = g01 * cwr.astype(jnp.bfloat16)
    contrib = jax.lax.dot_general(gw, y, (((0,), (0,)), ((), ())),
                                  preferred_element_type=jnp.float32)

    @pl.when(j == 0)
    def _():
        out_ref[...] = contrib

    @pl.when(j != 0)
    def _():
        out_ref[...] = out_ref[...] + contrib


def _moe_dispatch(tile_map, tile_base, h, cw, rk, w1, b1, w2, b2):
    grid_spec = pltpu.PrefetchScalarGridSpec(
        num_scalar_prefetch=2,
        grid=(NT,),
        in_specs=[
            pl.BlockSpec((N, D), lambda j, m, bs: (0, 0)),
            pl.BlockSpec((E, N), lambda j, m, bs: (0, 0)),
            pl.BlockSpec((E, N), lambda j, m, bs: (0, 0)),
            pl.BlockSpec((1, D, F), lambda j, m, bs: (m[j], 0, 0)),
            pl.BlockSpec((1, 1, F), lambda j, m, bs: (m[j], 0, 0)),
            pl.BlockSpec((1, F, D), lambda j, m, bs: (m[j], 0, 0)),
            pl.BlockSpec((1, 1, D), lambda j, m, bs: (m[j], 0, 0)),
        ],
        out_specs=pl.BlockSpec((N, D), lambda j, m, bs: (0, 0)),
    )
    return pl.pallas_call(
        _moe_body,
        grid_spec=grid_spec,
        out_shape=jax.ShapeDtypeStruct((N, D), jnp.float32),
        interpret=_INTERPRET,
    )(tile_map, tile_base, h, cw, rk, w1, b1, w2, b2)


def _moe_block(x, p):
    h, cw, rk, cnt = _router(x, p['ln_g'].reshape(1, D),
                             p['ln_b'].reshape(1, D), p['gate'])
    counts = cnt.reshape(E).astype(jnp.int32)
    tiles = (counts + (T_MOE - 1)) // T_MOE
    tend = jnp.cumsum(tiles)
    tstart = tend - tiles
    j = jnp.arange(NT, dtype=jnp.int32)
    e_of = jnp.sum((j[:, None] >= tend[None, :]).astype(jnp.int32), axis=1)
    valid = e_of < E
    e_clamped = jnp.where(valid, e_of, 0)
    tile_map = e_clamped.astype(jnp.int32)
    tile_base = jnp.where(
        valid, (j - tstart[e_clamped]) * T_MOE, jnp.int32(1 << 20)
    ).astype(jnp.int32)
    return x + _moe_dispatch(tile_map, tile_base, h, cw, rk,
                             p['w1'].astype(jnp.bfloat16),
                             p['b1'].reshape(E, 1, F),
                             p['w2'].astype(jnp.bfloat16),
                             p['b2'].reshape(E, 1, D))


def _head_body(x_ref, g_ref, b_ref, w_ref, hb_ref, out_ref):
    h = _ln_in(x_ref[...], g_ref[...], b_ref[...])
    out_ref[...] = jnp.dot(h, w_ref[...],
                           preferred_element_type=jnp.float32) + hb_ref[...]


def _head(x, g, b, w, hb):
    TM = 256
    VO = VOCAB - 1
    return pl.pallas_call(
        _head_body,
        grid=(N // TM,),
        in_specs=[
            pl.BlockSpec((TM, D), lambda i: (i, 0)),
            pl.BlockSpec((1, D), lambda i: (0, 0)),
            pl.BlockSpec((1, D), lambda i: (0, 0)),
            pl.BlockSpec((D, VO), lambda i: (0, 0)),
            pl.BlockSpec((1, VO), lambda i: (0, 0)),
        ],
        out_specs=pl.BlockSpec((TM, VO), lambda i: (i, 0)),
        out_shape=jax.ShapeDtypeStruct((N, VO), jnp.float32),
        interpret=_INTERPRET,
    )(x, g, b, w, hb)


def _dense_block(x, p):
    qkv = _ln_qkv(x, p['ln1_g'].reshape(1, D), p['ln1_b'].reshape(1, D),
                  p['qkv'].astype(jnp.bfloat16))
    o = _attention(qkv)
    return _proj_mlp(x, o, p['proj'].astype(jnp.bfloat16),
                     p['ln2_g'].reshape(1, D),
                     p['ln2_b'].reshape(1, D), p['fc1'].astype(jnp.bfloat16),
                     p['b1'].reshape(1, F), p['fc2'].astype(jnp.bfloat16),
                     p['b2'].reshape(1, D))


def kernel(tokens, time_t, perturb_id, params):
    freqs = jnp.arange(MAXFREQ, dtype=jnp.float32)
    angles = time_t[:, None] * freqs[None, :] * math.pi
    fourier = jnp.concatenate([jnp.sin(angles), jnp.cos(angles)], axis=-1)
    table_p = jnp.pad(params['token_emb'], ((0, 128 - VOCAB), (0, 0)))
    pert_row = params['pert_base'][perturb_id]
    la_row = params['loraA'][perturb_id]
    tok2d = tokens.reshape(N, 1).astype(jnp.int32)
    x = _embed(tok2d, table_p, fourier, params['time_W'],
               params['time_b'].reshape(1, D), pert_row, la_row,
               params['loraB'])
    for i in range(max(NDENSE, NMOE)):
        if i < NDENSE:
            x = _dense_block(x, params['dense'][i])
        if i < NMOE:
            x = _moe_block(x, params['moe'][i])
    out = _head(x, params['lnf_g'].reshape(1, D), params['lnf_b'].reshape(1, D),
                params['head_W'], params['head_b'].reshape(1, VOCAB - 1))
    return out.reshape(B, N, VOCAB - 1)


# speedup vs baseline: 1.1469x; 1.1392x over previous
import functools
import math

import jax
import jax.numpy as jnp
from jax.experimental import pallas as pl
from jax.experimental.pallas import tpu as pltpu

B, N = 1, 2048
D, H = 768, 12
HD = D // H
E, MULT = 8, 4
F = MULT * D
VOCAB, NPERT, LORA = 65, 10000, 32
MAXFREQ = 64
NDENSE, NMOE = 2, 2

T_MOE = 256
NT = (2 * N) // T_MOE + E

_INTERPRET = False


def _gelu(x):
    return jax.nn.gelu(x, approximate=True)


def _ln_in(x, g, b):
    m = jnp.mean(x, axis=-1, keepdims=True)
    xm = x - m
    v = jnp.mean(xm * xm, axis=-1, keepdims=True)
    return xm / jnp.sqrt(v + 1e-5) * g + b


def _embed_body(tok_ref, tab_ref, fourier_ref, tw_ref, tb_ref, pert_ref,
                la_ref, lb_ref, out_ref):
    hp = jax.lax.Precision.HIGHEST
    bias = (jnp.dot(fourier_ref[...], tw_ref[...], precision=hp,
                    preferred_element_type=jnp.float32)
            + tb_ref[...] + pert_ref[...]
            + jnp.dot(la_ref[...], lb_ref[...], precision=hp,
                      preferred_element_type=jnp.float32))
    lanes = jax.lax.broadcasted_iota(jnp.int32, (N, 128), 1)
    onehot = (tok_ref[...] == lanes).astype(jnp.float32)
    out_ref[...] = jnp.dot(onehot, tab_ref[...], precision=hp,
                           preferred_element_type=jnp.float32) + bias


def _embed(tokens, table_p, fourier, time_W, time_b, pert_row, la_row, loraB):
    return pl.pallas_call(
        _embed_body,
        out_shape=jax.ShapeDtypeStruct((N, D), jnp.float32),
        interpret=_INTERPRET,
    )(tokens, table_p, fourier, time_W, time_b, pert_row, la_row, loraB)


def _ln_qkv_body(x_ref, g_ref, b_ref, w_ref, out_ref):
    h = _ln_in(x_ref[...], g_ref[...], b_ref[...])
    out_ref[...] = jnp.dot(h, w_ref[...], preferred_element_type=jnp.float32)


def _ln_qkv(x, g, b, w):
    TM = 256
    return pl.pallas_call(
        _ln_qkv_body,
        grid=(N // TM,),
        in_specs=[
            pl.BlockSpec((TM, D), lambda i: (i, 0)),
            pl.BlockSpec((1, D), lambda i: (0, 0)),
            pl.BlockSpec((1, D), lambda i: (0, 0)),
            pl.BlockSpec((D, 3 * D), lambda i: (0, 0)),
        ],
        out_specs=pl.BlockSpec((TM, 3 * D), lambda i: (i, 0)),
        out_shape=jax.ShapeDtypeStruct((N, 3 * D), jnp.float32),
        interpret=_INTERPRET,
    )(x, g, b, w)


def _attn_body(q_ref, k_ref, v_ref, o_ref):
    q = q_ref[0] * (HD ** -0.5)
    s = jax.lax.dot_general(q, k_ref[0], (((1,), (1,)), ((), ())),
                            preferred_element_type=jnp.float32)
    m = jnp.max(s, axis=1, keepdims=True)
    p = jnp.exp(s - m)
    p = p / jnp.sum(p, axis=1, keepdims=True)
    o_ref[0] = jnp.dot(p, v_ref[0], preferred_element_type=jnp.float32)


def _attention(qkv):
    qkv_t = qkv.reshape(N, 3 * H, HD).transpose(1, 0, 2)
    o = pl.pallas_call(
        _attn_body,
        grid=(H,),
        in_specs=[
            pl.BlockSpec((1, N, HD), lambda h: (h, 0, 0)),
            pl.BlockSpec((1, N, HD), lambda h: (H + h, 0, 0)),
            pl.BlockSpec((1, N, HD), lambda h: (2 * H + h, 0, 0)),
        ],
        out_specs=pl.BlockSpec((1, N, HD), lambda h: (h, 0, 0)),
        out_shape=jax.ShapeDtypeStruct((H, N, HD), jnp.float32),
        interpret=_INTERPRET,
    )(qkv_t, qkv_t, qkv_t)
    return o.transpose(1, 0, 2).reshape(N, D)


def _proj_mlp_body(x_ref, o_ref, pw_ref, g_ref, b_ref, fc1_ref, b1_ref,
                   fc2_ref, b2_ref, out_ref):
    x2 = x_ref[...] + jnp.dot(o_ref[...], pw_ref[...],
                              preferred_element_type=jnp.float32)
    h2 = _ln_in(x2, g_ref[...], b_ref[...])
    a = _gelu(jnp.dot(h2, fc1_ref[...], preferred_element_type=jnp.float32)
              + b1_ref[...])
    y = jnp.dot(a, fc2_ref[...], preferred_element_type=jnp.float32) + b2_ref[...]
    out_ref[...] = x2 + y


def _proj_mlp(x, o, proj, g, b, fc1, b1, fc2, b2):
    TM = 256
    return pl.pallas_call(
        _proj_mlp_body,
        grid=(N // TM,),
        in_specs=[
            pl.BlockSpec((TM, D), lambda i: (i, 0)),
            pl.BlockSpec((TM, D), lambda i: (i, 0)),
            pl.BlockSpec((D, D), lambda i: (0, 0)),
            pl.BlockSpec((1, D), lambda i: (0, 0)),
            pl.BlockSpec((1, D), lambda i: (0, 0)),
            pl.BlockSpec((D, F), lambda i: (0, 0)),
            pl.BlockSpec((1, F), lambda i: (0, 0)),
            pl.BlockSpec((F, D), lambda i: (0, 0)),
            pl.BlockSpec((1, D), lambda i: (0, 0)),
        ],
        out_specs=pl.BlockSpec((TM, D), lambda i: (i, 0)),
        out_shape=jax.ShapeDtypeStruct((N, D), jnp.float32),
        interpret=_INTERPRET,
    )(x, o, proj, g, b, fc1, b1, fc2, b2)


def _router_body(x_ref, g_ref, b_ref, gate_ref, h_ref, cw_ref, rk_ref,
                 cnt_ref):
    h = _ln_in(x_ref[...], g_ref[...], b_ref[...])
    h_ref[...] = h
    lg = jnp.dot(h, gate_ref[...], preferred_element_type=jnp.float32)
    m = jnp.max(lg, axis=1, keepdims=True)
    p = jnp.exp(lg - m)
    p = p / jnp.sum(p, axis=1, keepdims=True)
    eio = jax.lax.broadcasted_iota(jnp.int32, (N, E), 1)
    v1 = jnp.max(p, axis=1, keepdims=True)
    i1 = jnp.min(jnp.where(p == v1, eio, E + 1), axis=1, keepdims=True)
    pm = jnp.where(eio == i1, -jnp.inf, p)
    v2 = jnp.max(pm, axis=1, keepdims=True)
    i2 = jnp.min(jnp.where(pm == v2, eio, E + 1), axis=1, keepdims=True)
    denom = v1 + v2 + 1e-9
    cwn = (jnp.where(eio == i1, v1 / denom, 0.0)
           + jnp.where(eio == i2, v2 / denom, 0.0))
    cw = cwn.T
    cw_ref[...] = cw
    ind = ((eio == i1) | (eio == i2)).astype(jnp.float32).T
    acc = ind
    s = 1
    while s < N:
        shifted = jnp.pad(acc, ((0, 0), (s, 0)))[:, :N]
        acc = acc + shifted
        s *= 2
    excl = acc - ind
    rk_ref[...] = jnp.where(ind > 0.5, excl, -1.0)
    cnt_ref[...] = jnp.sum(ind, axis=1, keepdims=True)


def _router(x, g, b, gate):
    return pl.pallas_call(
        _router_body,
        out_shape=(
            jax.ShapeDtypeStruct((N, D), jnp.float32),
            jax.ShapeDtypeStruct((E, N), jnp.float32),
            jax.ShapeDtypeStruct((E, N), jnp.float32),
            jax.ShapeDtypeStruct((E, 1), jnp.float32),
        ),
        interpret=_INTERPRET,
    )(x, g, b, gate)


def _moe_body(map_ref, base_ref, h_ref, cw_ref, rk_ref, w1_ref,
              b1_ref, w2_ref, b2_ref, out_ref):
    j = pl.program_id(0)
    e = map_ref[j]
    base = base_ref[j].astype(jnp.float32)
    rk = rk_ref[pl.ds(e, 1), :]
    cwr = cw_ref[pl.ds(e, 1), :]
    rows = jax.lax.broadcasted_iota(
        jnp.int32, (T_MOE, 1), 0).astype(jnp.float32) + base
    g01 = (rk == rows).astype(jnp.float32)
    ht = jnp.dot(g01, h_ref[...], preferred_element_type=jnp.float32)
    a = _gelu(jnp.dot(ht, w1_ref[0], preferred_element_type=jnp.float32)
              + b1_ref[0])
    y = jnp.dot(a, w2_ref[0], preferred_element_type=jnp.float32) + b2_ref[0]
    gw = g01 * cwr
    contrib = jax.lax.dot_general(gw, y, (((0,), (0,)), ((), ())),
                                  preferred_element_type=jnp.float32)

    @pl.when(j == 0)
    def _():
        out_ref[...] = contrib

    @pl.when(j != 0)
    def _():
        out_ref[...] = out_ref[...] + contrib


def _moe_dispatch(tile_map, tile_base, h, cw, rk, w1, b1, w2, b2):
    grid_spec = pltpu.PrefetchScalarGridSpec(
        num_scalar_prefetch=2,
        grid=(NT,),
        in_specs=[
            pl.BlockSpec((N, D), lambda j, m, bs: (0, 0)),
            pl.BlockSpec((E, N), lambda j, m, bs: (0, 0)),
            pl.BlockSpec((E, N), lambda j, m, bs: (0, 0)),
            pl.BlockSpec((1, D, F), lambda j, m, bs: (m[j], 0, 0)),
            pl.BlockSpec((1, 1, F), lambda j, m, bs: (m[j], 0, 0)),
            pl.BlockSpec((1, F, D), lambda j, m, bs: (m[j], 0, 0)),
            pl.BlockSpec((1, 1, D), lambda j, m, bs: (m[j], 0, 0)),
        ],
        out_specs=pl.BlockSpec((N, D), lambda j, m, bs: (0, 0)),
    )
    return pl.pallas_call(
        _moe_body,
        grid_spec=grid_spec,
        out_shape=jax.ShapeDtypeStruct((N, D), jnp.float32),
        interpret=_INTERPRET,
    )(tile_map, tile_base, h, cw, rk, w1, b1, w2, b2)


def _moe_block(x, p):
    h, cw, rk, cnt = _router(x, p['ln_g'].reshape(1, D),
                             p['ln_b'].reshape(1, D), p['gate'])
    counts = cnt.reshape(E).astype(jnp.int32)
    tiles = (counts + (T_MOE - 1)) // T_MOE
    tend = jnp.cumsum(tiles)
    tstart = tend - tiles
    j = jnp.arange(NT, dtype=jnp.int32)
    e_of = jnp.sum((j[:, None] >= tend[None, :]).astype(jnp.int32), axis=1)
    valid = e_of < E
    e_clamped = jnp.where(valid, e_of, 0)
    tile_map = e_clamped.astype(jnp.int32)
    tile_base = jnp.where(
        valid, (j - tstart[e_clamped]) * T_MOE, jnp.int32(1 << 20)
    ).astype(jnp.int32)
    return x + _moe_dispatch(tile_map, tile_base, h, cw, rk,
                             p['w1'], p['b1'].reshape(E, 1, F),
                             p['w2'], p['b2'].reshape(E, 1, D))


def _head_body(x_ref, g_ref, b_ref, w_ref, hb_ref, out_ref):
    h = _ln_in(x_ref[...], g_ref[...], b_ref[...])
    out_ref[...] = jnp.dot(h, w_ref[...],
                           preferred_element_type=jnp.float32) + hb_ref[...]


def _head(x, g, b, w, hb):
    TM = 256
    VO = VOCAB - 1
    return pl.pallas_call(
        _head_body,
        grid=(N // TM,),
        in_specs=[
            pl.BlockSpec((TM, D), lambda i: (i, 0)),
            pl.BlockSpec((1, D), lambda i: (0, 0)),
            pl.BlockSpec((1, D), lambda i: (0, 0)),
            pl.BlockSpec((D, VO), lambda i: (0, 0)),
            pl.BlockSpec((1, VO), lambda i: (0, 0)),
        ],
        out_specs=pl.BlockSpec((TM, VO), lambda i: (i, 0)),
        out_shape=jax.ShapeDtypeStruct((N, VO), jnp.float32),
        interpret=_INTERPRET,
    )(x, g, b, w, hb)


def _dense_block(x, p):
    qkv = _ln_qkv(x, p['ln1_g'].reshape(1, D), p['ln1_b'].reshape(1, D),
                  p['qkv'])
    o = _attention(qkv)
    return _proj_mlp(x, o, p['proj'], p['ln2_g'].reshape(1, D),
                     p['ln2_b'].reshape(1, D), p['fc1'],
                     p['b1'].reshape(1, F), p['fc2'], p['b2'].reshape(1, D))


def kernel(tokens, time_t, perturb_id, params):
    freqs = jnp.arange(MAXFREQ, dtype=jnp.float32)
    angles = time_t[:, None] * freqs[None, :] * math.pi
    fourier = jnp.concatenate([jnp.sin(angles), jnp.cos(angles)], axis=-1)
    table_p = jnp.pad(params['token_emb'], ((0, 128 - VOCAB), (0, 0)))
    pert_row = params['pert_base'][perturb_id]
    la_row = params['loraA'][perturb_id]
    tok2d = tokens.reshape(N, 1).astype(jnp.int32)
    x = _embed(tok2d, table_p, fourier, params['time_W'],
               params['time_b'].reshape(1, D), pert_row, la_row,
               params['loraB'])
    for i in range(max(NDENSE, NMOE)):
        if i < NDENSE:
            x = _dense_block(x, params['dense'][i])
        if i < NMOE:
            x = _moe_block(x, params['moe'][i])
    out = _head(x, params['lnf_g'].reshape(1, D), params['lnf_b'].reshape(1, D),
                params['head_W'], params['head_b'].reshape(1, VOCAB - 1))
    return out.reshape(B, N, VOCAB - 1)
